# baseline (speedup 1.0000x reference)
import functools

import jax
import jax.numpy as jnp
from jax import lax
from jax.experimental import pallas as pl
from jax.experimental.pallas import tpu as pltpu
from jax.experimental.pallas import tpu_sc as plsc

NUM_WORKERS = 32
CHUNK = 128


def _gather_kernel(n_chunks, chunk, d, per_w):
    mesh = plsc.VectorSubcoreMesh(core_axis_name="c", subcore_axis_name="s")

    @functools.partial(
        pl.kernel,
        mesh=mesh,
        out_type=jax.ShapeDtypeStruct((NUM_WORKERS * per_w, d), jnp.float32),
        scratch_types=[
            pltpu.VMEM((n_chunks, chunk), jnp.int32),
            pltpu.VMEM((2, chunk, d), jnp.float32),
            pltpu.SemaphoreType.DMA,
        ],
        compiler_params=pltpu.CompilerParams(use_tc_tiling_on_sc=False),
    )
    def emb(idx_hbm, tab_hbm, out_hbm, idx_v, rows_v, gsem):
        c = lax.axis_index("c")
        s = lax.axis_index("s")
        wid = s * 2 + c
        base = wid * per_w
        pltpu.sync_copy(idx_hbm.at[wid], idx_v)

        def body(g, _):
            buf = lax.rem(g, 2)
            cp = pltpu.async_copy(tab_hbm.at[idx_v.at[g]], rows_v.at[buf], gsem)
            cp.wait()
            pltpu.sync_copy(rows_v.at[buf], out_hbm.at[pl.ds(base + g * chunk, chunk)])
            return 0

        lax.fori_loop(0, n_chunks, body, 0)

    return emb


def kernel(indices, table):
    b, sq = indices.shape
    v, d = table.shape
    n = b * sq
    per_w = n // NUM_WORKERS
    n_chunks = per_w // CHUNK
    idx = indices.reshape(NUM_WORKERS, n_chunks, CHUNK).astype(jnp.int32)
    out = _gather_kernel(n_chunks, CHUNK, d, per_w)(idx, table)
    return out.reshape(b, sq, d)


# speedup vs baseline: 1.1166x; 1.1166x over previous
import functools

import jax
import jax.numpy as jnp
from jax import lax
from jax.experimental import pallas as pl
from jax.experimental.pallas import tpu as pltpu
from jax.experimental.pallas import tpu_sc as plsc

NUM_WORKERS = 32
CHUNK = 128
NBUF = 4


def _gather_kernel(n_chunks, chunk, d, per_w):
    mesh = plsc.VectorSubcoreMesh(core_axis_name="c", subcore_axis_name="s")

    @functools.partial(
        pl.kernel,
        mesh=mesh,
        out_type=jax.ShapeDtypeStruct((NUM_WORKERS * per_w, d), jnp.float32),
        scratch_types=(
            [pltpu.VMEM((n_chunks, chunk), jnp.int32),
             pltpu.VMEM((NBUF, chunk, d), jnp.float32)]
            + [pltpu.SemaphoreType.DMA] * (2 * NBUF)
        ),
        compiler_params=pltpu.CompilerParams(use_tc_tiling_on_sc=False),
    )
    def emb(idx_hbm, tab_hbm, out_hbm, idx_v, rows_v, *sems):
        gsem = sems[:NBUF]
        wsem = sems[NBUF:]
        c = lax.axis_index("c")
        s = lax.axis_index("s")
        wid = s * 2 + c
        base = wid * per_w
        pltpu.sync_copy(idx_hbm.at[wid], idx_v)

        def fire_gather(slot, b):
            pltpu.async_copy(tab_hbm.at[idx_v.at[slot]], rows_v.at[b], gsem[b])

        def wait_gather(slot, b):
            pltpu.make_async_copy(
                tab_hbm.at[idx_v.at[slot]], rows_v.at[b], gsem[b]).wait()

        def fire_write(slot, b):
            pltpu.async_copy(
                rows_v.at[b], out_hbm.at[pl.ds(base + slot * chunk, chunk)],
                wsem[b])

        def wait_write(slot, b):
            pltpu.make_async_copy(
                rows_v.at[b], out_hbm.at[pl.ds(base + slot * chunk, chunk)],
                wsem[b]).wait()

        def do_slot(slot, k, fire, wait_prev):
            b = k % NBUF
            pb = (k - 1) % NBUF
            wait_gather(slot, b)
            fire_write(slot, b)
            if wait_prev:
                wait_write(slot - 1, pb)
            if fire:
                fire_gather(slot + NBUF - 1, pb)

        for j in range(NBUF - 1):
            fire_gather(j, j)

        for k in range(NBUF):
            do_slot(k, k, fire=(k + NBUF - 1 < n_chunks), wait_prev=(k >= 1))

        n_rounds = n_chunks // NBUF

        def body(r, _):
            s0 = r * NBUF
            for k in range(NBUF):
                do_slot(s0 + k, k, fire=True, wait_prev=True)
            return 0

        lax.fori_loop(1, n_rounds - 1, body, 0)

        s0 = (n_rounds - 1) * NBUF
        for k in range(NBUF):
            do_slot(s0 + k, k, fire=(s0 + k + NBUF - 1 < n_chunks),
                    wait_prev=True)

        wait_write(n_chunks - 1, (n_chunks - 1) % NBUF)

    return emb


def kernel(indices, table):
    b, sq = indices.shape
    v, d = table.shape
    n = b * sq
    per_w = n // NUM_WORKERS
    n_chunks = per_w // CHUNK
    idx = indices.reshape(NUM_WORKERS, n_chunks, CHUNK).astype(jnp.int32)
    out = _gather_kernel(n_chunks, CHUNK, d, per_w)(idx, table)
    return out.reshape(b, sq, d)


# speedup vs baseline: 1.4848x; 1.3298x over previous
import functools

import jax
import jax.numpy as jnp
from jax import lax
from jax.experimental import pallas as pl
from jax.experimental.pallas import tpu as pltpu
from jax.experimental.pallas import tpu_sc as plsc

NUM_WORKERS = 32
CHUNK = 128
NBUF = 4


def _gather_kernel(n_chunks, chunk, d, per_w):
    mesh = plsc.VectorSubcoreMesh(core_axis_name="c", subcore_axis_name="s")

    @functools.partial(
        pl.kernel,
        mesh=mesh,
        out_type=jax.ShapeDtypeStruct((NUM_WORKERS * per_w, 2 * d), jnp.float32),
        scratch_types=(
            [pltpu.VMEM((n_chunks, chunk), jnp.int32),
             pltpu.VMEM((NBUF, chunk, d), jnp.float32)]
            + [pltpu.SemaphoreType.DMA] * (2 * NBUF)
        ),
        compiler_params=pltpu.CompilerParams(use_tc_tiling_on_sc=False),
    )
    def emb(idx_hbm, tab_hbm, out_hbm, idx_v, rows_v, *sems):
        gsem = sems[:NBUF]
        wsem = sems[NBUF:]
        c = lax.axis_index("c")
        s = lax.axis_index("s")
        wid = s * 2 + c
        base = wid * per_w
        pltpu.sync_copy(idx_hbm.at[wid], idx_v)

        def fire_gather(slot, b):
            pltpu.async_copy(tab_hbm.at[idx_v.at[slot]], rows_v.at[b], gsem[b])

        def wait_gather(slot, b):
            pltpu.make_async_copy(
                tab_hbm.at[idx_v.at[slot]], rows_v.at[b], gsem[b]).wait()

        def fire_write(slot, b):
            pltpu.async_copy(
                rows_v.at[b],
                out_hbm.at[pl.ds(base + slot * chunk, chunk), pl.ds(0, d)],
                wsem[b])

        def wait_write(slot, b):
            pltpu.make_async_copy(
                rows_v.at[b],
                out_hbm.at[pl.ds(base + slot * chunk, chunk), pl.ds(0, d)],
                wsem[b]).wait()

        def do_slot(slot, k, fire, wait_prev):
            b = k % NBUF
            pb = (k - 1) % NBUF
            wait_gather(slot, b)
            fire_write(slot, b)
            if wait_prev:
                wait_write(slot - 1, pb)
            if fire:
                fire_gather(slot + NBUF - 1, pb)

        for j in range(NBUF - 1):
            fire_gather(j, j)

        for k in range(NBUF):
            do_slot(k, k, fire=(k + NBUF - 1 < n_chunks), wait_prev=(k >= 1))

        n_rounds = n_chunks // NBUF

        def body(r, _):
            s0 = r * NBUF
            for k in range(NBUF):
                do_slot(s0 + k, k, fire=True, wait_prev=True)
            return 0

        lax.fori_loop(1, n_rounds - 1, body, 0)

        s0 = (n_rounds - 1) * NBUF
        for k in range(NBUF):
            do_slot(s0 + k, k, fire=(s0 + k + NBUF - 1 < n_chunks),
                    wait_prev=True)

        wait_write(n_chunks - 1, (n_chunks - 1) % NBUF)

    return emb


def kernel(indices, table):
    b, sq = indices.shape
    v, d = table.shape
    n = b * sq
    per_w = n // NUM_WORKERS
    n_chunks = per_w // CHUNK
    idx = indices.reshape(NUM_WORKERS, n_chunks, CHUNK).astype(jnp.int32)
    out = _gather_kernel(n_chunks, CHUNK, d, per_w)(idx, table)
    return out[:, :d].reshape(b, sq, d)
